# baseline (speedup 1.0000x reference)
import functools

import jax
import jax.numpy as jnp
from jax import lax
from jax.experimental import pallas as pl
from jax.experimental.pallas import tpu as pltpu
from jax.experimental.pallas import tpu_sc as plsc

B, S, H = 1, 2048, 1024
NH, DH = 16, 64
E, K = 8, 2
IM = 512
IS = 1024
EPS = 1e-6

BT = 256
GT = 256
NPAIR = S * K
NPAD = NPAIR + E * GT
NB = NPAD // GT
NEG = jnp.finfo(jnp.float32).min


def _norm_mm_body(x_ref, w_ref, wm_ref, o_ref):
    x = x_ref[...]
    nrm = lax.rsqrt(jnp.mean(x * x, axis=-1, keepdims=True) + EPS)
    xn = x * nrm * w_ref[...]
    o_ref[...] = jnp.dot(xn, wm_ref[...], preferred_element_type=jnp.float32)


def _norm_qkv(x, ln_w, wqkv):
    return pl.pallas_call(
        _norm_mm_body,
        grid=(S // BT,),
        in_specs=[
            pl.BlockSpec((BT, H), lambda i: (i, 0)),
            pl.BlockSpec((1, H), lambda i: (0, 0)),
            pl.BlockSpec((H, 3 * H), lambda i: (0, 0)),
        ],
        out_specs=pl.BlockSpec((BT, 3 * H), lambda i: (i, 0)),
        out_shape=jax.ShapeDtypeStruct((S, 3 * H), jnp.float32),
    )(x, ln_w, wqkv)


def _rope(t, c, s):
    t1 = t[:, : DH // 2]
    t2 = t[:, DH // 2 :]
    rot = jnp.concatenate([-t2, t1], axis=-1)
    return t * c + rot * s


def _attn_body(q_ref, k_ref, v_ref, cq_ref, sq_ref, ck_ref, sk_ref, o_ref):
    i = pl.program_id(1)
    row = i * BT + lax.broadcasted_iota(jnp.int32, (BT, S), 0)
    col = lax.broadcasted_iota(jnp.int32, (BT, S), 1)
    causal = col <= row
    for h in range(2):
        sl = slice(h * DH, (h + 1) * DH)
        q = _rope(q_ref[:, sl], cq_ref[...], sq_ref[...])
        k = _rope(k_ref[:, sl], ck_ref[...], sk_ref[...])
        scores = jax.lax.dot_general(
            q, k, (((1,), (1,)), ((), ())), preferred_element_type=jnp.float32
        ) * (1.0 / (DH ** 0.5))
        scores = jnp.where(causal, scores, NEG)
        m = jnp.max(scores, axis=-1, keepdims=True)
        p = jnp.exp(scores - m)
        l = jnp.sum(p, axis=-1, keepdims=True)
        o_ref[:, sl] = (
            jnp.dot(p, v_ref[:, sl], preferred_element_type=jnp.float32) / l
        )


def _attention(qkv, cos, sin):
    return pl.pallas_call(
        _attn_body,
        grid=(NH // 2, S // BT),
        in_specs=[
            pl.BlockSpec((BT, 2 * DH), lambda h, i: (i, h)),
            pl.BlockSpec((S, 2 * DH), lambda h, i: (0, NH // 2 + h)),
            pl.BlockSpec((S, 2 * DH), lambda h, i: (0, NH + h)),
            pl.BlockSpec((BT, DH), lambda h, i: (i, 0)),
            pl.BlockSpec((BT, DH), lambda h, i: (i, 0)),
            pl.BlockSpec((S, DH), lambda h, i: (0, 0)),
            pl.BlockSpec((S, DH), lambda h, i: (0, 0)),
        ],
        out_specs=pl.BlockSpec((BT, 2 * DH), lambda h, i: (i, h)),
        out_shape=jax.ShapeDtypeStruct((S, H), jnp.float32),
    )(qkv, qkv, qkv, cos, sin, cos, sin)


def _mm_res_body(x_ref, w_ref, r_ref, o_ref):
    o_ref[...] = (
        jnp.dot(x_ref[...], w_ref[...], preferred_element_type=jnp.float32)
        + r_ref[...]
    )


def _mm_residual(x, w, resid):
    n = w.shape[1]
    return pl.pallas_call(
        _mm_res_body,
        grid=(S // BT,),
        in_specs=[
            pl.BlockSpec((BT, x.shape[1]), lambda i: (i, 0)),
            pl.BlockSpec((x.shape[1], n), lambda i: (0, 0)),
            pl.BlockSpec((BT, n), lambda i: (i, 0)),
        ],
        out_specs=pl.BlockSpec((BT, n), lambda i: (i, 0)),
        out_shape=jax.ShapeDtypeStruct((S, n), jnp.float32),
    )(x, w, resid)


def _norm_router_body(x_ref, w_ref, g_ref, h_ref, r_ref):
    x = x_ref[...]
    nrm = lax.rsqrt(jnp.mean(x * x, axis=-1, keepdims=True) + EPS)
    xn = x * nrm * w_ref[...]
    h_ref[...] = xn
    logits = jnp.dot(xn, g_ref[...], preferred_element_type=jnp.float32)
    col = lax.broadcasted_iota(jnp.int32, (BT, 128), 1)
    colf = col.astype(jnp.float32)
    lg = jnp.where(col < E, logits, NEG)
    m1 = jnp.max(lg, axis=-1, keepdims=True)
    e1 = jnp.min(jnp.where(lg == m1, colf, 1e9), axis=-1, keepdims=True)
    lg2 = jnp.where(colf == e1, NEG, lg)
    m2 = jnp.max(lg2, axis=-1, keepdims=True)
    e2 = jnp.min(jnp.where(lg2 == m2, colf, 1e9), axis=-1, keepdims=True)
    w1 = 1.0 / (1.0 + jnp.exp(m2 - m1))
    w2 = 1.0 - w1
    r_ref[...] = (
        w1 * (col == 0) + w2 * (col == 1) + e1 * (col == 2) + e2 * (col == 3)
    )


def _norm_router(x, ln_w, gate_pad):
    return pl.pallas_call(
        _norm_router_body,
        grid=(S // BT,),
        in_specs=[
            pl.BlockSpec((BT, H), lambda i: (i, 0)),
            pl.BlockSpec((1, H), lambda i: (0, 0)),
            pl.BlockSpec((H, 128), lambda i: (0, 0)),
        ],
        out_specs=[
            pl.BlockSpec((BT, H), lambda i: (i, 0)),
            pl.BlockSpec((BT, 128), lambda i: (i, 0)),
        ],
        out_shape=[
            jax.ShapeDtypeStruct((S, H), jnp.float32),
            jax.ShapeDtypeStruct((S, 128), jnp.float32),
        ],
    )(x, ln_w, gate_pad)


def _make_sc_gather(n_rows, n_cols, chunk):
    info = plsc.get_sparse_core_info()
    nc, ns = info.num_cores, info.num_subcores
    nw = nc * ns
    assert n_rows % (nw * chunk) == 0
    b_per_w = n_rows // nw
    n_chunks = b_per_w // chunk
    mesh = plsc.VectorSubcoreMesh(core_axis_name="c", subcore_axis_name="s")

    def body(table_hbm, idx_hbm, out_hbm, idx_v, rows_v, sem):
        wid = lax.axis_index("s") * nc + lax.axis_index("c")
        base = wid * b_per_w
        for c in range(n_chunks):
            pltpu.sync_copy(idx_hbm.at[pl.ds(base + c * chunk, chunk)], idx_v)
            pltpu.async_copy(table_hbm.at[idx_v], rows_v, sem).wait()
            pltpu.sync_copy(rows_v, out_hbm.at[pl.ds(base + c * chunk, chunk)])

    return functools.partial(
        pl.kernel,
        mesh=mesh,
        out_type=jax.ShapeDtypeStruct((n_rows, n_cols), jnp.float32),
        scratch_types=[
            pltpu.VMEM((chunk,), jnp.int32),
            pltpu.VMEM((chunk, n_cols), jnp.float32),
            pltpu.SemaphoreType.DMA,
        ],
    )(body)


def _gmm_body(be_ref, x_ref, w13_ref, w2_ref, ws_ref, o_ref):
    x = x_ref[...]
    gu = jnp.dot(x, w13_ref[0], preferred_element_type=jnp.float32)
    g = gu[:, :IM]
    u = gu[:, IM:]
    a = (g / (1.0 + jnp.exp(-g))) * u
    y = jnp.dot(a, w2_ref[0], preferred_element_type=jnp.float32)
    o_ref[...] = y * ws_ref[0, 0][:, None]


def _grouped_gemm(xs, w13, w2, w_sorted, block_expert):
    grid_spec = pltpu.PrefetchScalarGridSpec(
        num_scalar_prefetch=1,
        grid=(NB,),
        in_specs=[
            pl.BlockSpec((GT, H), lambda b, be: (b, 0)),
            pl.BlockSpec((1, H, 2 * IM), lambda b, be: (be[b], 0, 0)),
            pl.BlockSpec((1, IM, H), lambda b, be: (be[b], 0, 0)),
            pl.BlockSpec((1, 1, GT), lambda b, be: (b, 0, 0)),
        ],
        out_specs=pl.BlockSpec((GT, H), lambda b, be: (b, 0)),
    )
    return pl.pallas_call(
        _gmm_body,
        grid_spec=grid_spec,
        out_shape=jax.ShapeDtypeStruct((NPAD, H), jnp.float32),
    )(block_expert, xs, w13, w2, w_sorted.reshape(NB, 1, GT))


def _swiglu_body(x_ref, w_ref, o_ref):
    gu = jnp.dot(x_ref[...], w_ref[...], preferred_element_type=jnp.float32)
    g = gu[:, :IS]
    u = gu[:, IS:]
    o_ref[...] = (g / (1.0 + jnp.exp(-g))) * u


def _shared_up(h2, w_shared):
    return pl.pallas_call(
        _swiglu_body,
        grid=(S // BT,),
        in_specs=[
            pl.BlockSpec((BT, H), lambda i: (i, 0)),
            pl.BlockSpec((H, 2 * IS), lambda i: (0, 0)),
        ],
        out_specs=pl.BlockSpec((BT, IS), lambda i: (i, 0)),
        out_shape=jax.ShapeDtypeStruct((S, IS), jnp.float32),
    )(h2, w_shared)


def _final_body(a_ref, w_ref, r_ref, y_ref, o_ref):
    y = y_ref[...]
    o_ref[...] = (
        jnp.dot(a_ref[...], w_ref[...], preferred_element_type=jnp.float32)
        + r_ref[...]
        + y[:, :H]
        + y[:, H:]
    )


def _final(act, sh_down, resid, yt):
    return pl.pallas_call(
        _final_body,
        grid=(S // BT,),
        in_specs=[
            pl.BlockSpec((BT, IS), lambda i: (i, 0)),
            pl.BlockSpec((IS, H), lambda i: (0, 0)),
            pl.BlockSpec((BT, H), lambda i: (i, 0)),
            pl.BlockSpec((BT, 2 * H), lambda i: (i, 0)),
        ],
        out_specs=pl.BlockSpec((BT, H), lambda i: (i, 0)),
        out_shape=jax.ShapeDtypeStruct((S, H), jnp.float32),
    )(act, sh_down, resid, yt)


def kernel(hidden_states, ln1_w, ln2_w, Wq, Wk, Wv, Wo, gate_w, W13, W2,
           sh_gate, sh_up, sh_down, cos, sin):
    x = hidden_states.reshape(S, H)
    wqkv = jnp.concatenate([Wq, Wk, Wv], axis=1)
    w_shared = jnp.concatenate([sh_gate, sh_up], axis=1)
    gate_pad = jnp.pad(gate_w.T, ((0, 0), (0, 128 - E)))
    ln1 = ln1_w.reshape(1, H)
    ln2 = ln2_w.reshape(1, H)

    qkv = _norm_qkv(x, ln1, wqkv)
    attn = _attention(qkv, cos, sin)
    hidden2 = _mm_residual(attn, Wo, x)

    h2, rinfo = _norm_router(hidden2, ln2, gate_pad)
    w1 = rinfo[:, 0]
    w2r = rinfo[:, 1]
    e1 = rinfo[:, 2].astype(jnp.int32)
    e2 = rinfo[:, 3].astype(jnp.int32)

    flat_e = jnp.stack([e1, e2], axis=1).reshape(NPAIR)
    flat_w = jnp.stack([w1, w2r], axis=1).reshape(NPAIR)
    sort_idx = jnp.argsort(flat_e, stable=True).astype(jnp.int32)
    sorted_e = flat_e[sort_idx]
    counts = jnp.bincount(flat_e, length=E)
    start = jnp.concatenate([jnp.zeros((1,), counts.dtype),
                             jnp.cumsum(counts)[:-1]])
    padded_counts = ((counts + GT - 1) // GT) * GT
    pstart = jnp.concatenate([jnp.zeros((1,), counts.dtype),
                              jnp.cumsum(padded_counts)[:-1]])
    i_ar = jnp.arange(NPAIR)
    pos = (pstart[sorted_e] + (i_ar - start[sorted_e])).astype(jnp.int32)
    dispatch_idx = jnp.zeros((NPAD,), jnp.int32).at[pos].set(sort_idx // K)
    w_sorted = jnp.zeros((NPAD,), jnp.float32).at[pos].set(flat_w[sort_idx])
    inv_pos = jnp.zeros((NPAIR,), jnp.int32).at[sort_idx].set(pos)
    cum_p = jnp.cumsum(padded_counts)
    block_expert = jnp.minimum(
        jnp.searchsorted(cum_p, jnp.arange(NB) * GT, side="right"), E - 1
    ).astype(jnp.int32)

    xs = _make_sc_gather(NPAD, H, 32)(h2, dispatch_idx)

    ys = _grouped_gemm(xs, W13, W2, w_sorted, block_expert)

    yt = _make_sc_gather(NPAIR, H, 32)(ys, inv_pos)

    act = _shared_up(h2, w_shared)
    out = _final(act, sh_down, hidden2, yt.reshape(S, 2 * H))
    return out.reshape(B, S, H)
